# baseline (speedup 1.0000x reference)
import functools

import jax
import jax.numpy as jnp
from jax import lax
from jax.experimental import pallas as pl
from jax.experimental.pallas import tpu as pltpu
from jax.experimental.pallas import tpu_sc as plsc

B, C, H, W = 4, 384, 224, 224
HW = H * W
K = 8192
HID = 256

ROWS = B * C
ROW_BLK = 128
COL_BLK = 7168
N_ROW = ROWS // ROW_BLK
N_COL = HW // COL_BLK

SC_CORES = 2
SC_SUBCORES = 16
SC_LANES = 16


def _sum_body(x_ref, out_ref):
    j = pl.program_id(1)
    part = jnp.sum(x_ref[...], axis=1, keepdims=True)

    @pl.when(j == 0)
    def _():
        out_ref[...] = part

    @pl.when(j > 0)
    def _():
        out_ref[...] += part

    @pl.when(j == N_COL - 1)
    def _():
        out_ref[...] = out_ref[...] / jnp.float32(HW)


def _summary(x2):
    return pl.pallas_call(
        _sum_body,
        grid=(N_ROW, N_COL),
        in_specs=[pl.BlockSpec((ROW_BLK, COL_BLK), lambda i, j: (i, j))],
        out_specs=pl.BlockSpec((ROW_BLK, 1), lambda i, j: (i, 0)),
        out_shape=jax.ShapeDtypeStruct((ROWS, 1), jnp.float32),
        compiler_params=pltpu.CompilerParams(
            dimension_semantics=("arbitrary", "arbitrary")),
    )(x2)


K_BLK = 1024
N_K = K // K_BLK


def _d2_body(s_ref, p_ref, out_ref):
    s = s_ref[...]
    p = p_ref[...]
    ssq = jnp.sum(s * s, axis=1, keepdims=True)
    ones = jnp.ones((1, C), jnp.float32)
    psq = lax.dot_general(ones, p * p, (((1,), (1,)), ((), ())),
                          preferred_element_type=jnp.float32,
                          precision=lax.Precision.HIGHEST)
    cross = lax.dot_general(s, p, (((1,), (1,)), ((), ())),
                            preferred_element_type=jnp.float32,
                            precision=lax.Precision.HIGHEST)
    out_ref[...] = (ssq + psq) - 2.0 * cross


def _distances(summary, prototypes):
    return pl.pallas_call(
        _d2_body,
        grid=(N_K,),
        in_specs=[
            pl.BlockSpec((B, C), lambda j: (0, 0)),
            pl.BlockSpec((K_BLK, C), lambda j: (j, 0)),
        ],
        out_specs=pl.BlockSpec((B, K_BLK), lambda j: (0, j)),
        out_shape=jax.ShapeDtypeStruct((B, K), jnp.float32),
    )(summary, prototypes)


def _argmin_gather_body(d2_hbm, protos_hbm, out_hbm, d2_v, idx_v, rows_v, sem):
    cid = lax.axis_index("c")
    sid = lax.axis_index("s")
    wid = sid * SC_CORES + cid

    @pl.when(wid == 0)
    def _():
        pltpu.sync_copy(d2_hbm, d2_v)
        iota = lax.iota(jnp.int32, SC_LANES)
        idxvec = jnp.zeros((SC_LANES,), jnp.int32)
        for b in range(B):
            def body(i, carry):
                best, bidx = carry
                v = d2_v[b, pl.ds(i * SC_LANES, SC_LANES)]
                cand = i * SC_LANES + iota
                upd = v < best
                return (jnp.where(upd, v, best), jnp.where(upd, cand, bidx))

            best, bidx = lax.fori_loop(
                0, K // SC_LANES, body,
                (jnp.full((SC_LANES,), jnp.inf, jnp.float32),
                 jnp.zeros((SC_LANES,), jnp.int32)))
            _, sv = plsc.sort_key_val(best, bidx)
            idxvec = jnp.where(iota == b, sv[0], idxvec)
        idx_v[...] = idxvec
        pltpu.async_copy(protos_hbm.at[idx_v], rows_v, sem).wait()
        pltpu.sync_copy(rows_v.at[pl.ds(0, B)], out_hbm)


def _argmin_gather(d2, prototypes):
    mesh = plsc.VectorSubcoreMesh(core_axis_name="c", subcore_axis_name="s",
                                  num_cores=SC_CORES, num_subcores=SC_SUBCORES)
    fn = pl.kernel(
        _argmin_gather_body,
        out_type=jax.ShapeDtypeStruct((B, C), jnp.float32),
        mesh=mesh,
        scratch_types=[
            pltpu.VMEM((B, K), jnp.float32),
            pltpu.VMEM((SC_LANES,), jnp.int32),
            pltpu.VMEM((SC_LANES, C), jnp.float32),
            pltpu.SemaphoreType.DMA,
        ],
        compiler_params=pltpu.CompilerParams(needs_layout_passes=False),
    )
    return fn(d2, prototypes)


def _mlp_body(m_ref, w1_ref, b1_ref, w2_ref, b2_ref, out_ref):
    m = m_ref[...]
    h = lax.dot_general(m, w1_ref[...], (((1,), (1,)), ((), ())),
                        preferred_element_type=jnp.float32,
                        precision=lax.Precision.HIGHEST) + b1_ref[...]
    h = jnp.maximum(h, 0.0)
    o = lax.dot_general(h, w2_ref[...], (((1,), (1,)), ((), ())),
                        preferred_element_type=jnp.float32,
                        precision=lax.Precision.HIGHEST) + b2_ref[...]
    out_ref[...] = jax.nn.sigmoid(o)


def _mlp(matched, W1, b1, W2, b2):
    return pl.pallas_call(
        _mlp_body,
        out_shape=jax.ShapeDtypeStruct((B, C), jnp.float32),
    )(matched, W1, b1.reshape(1, HID), W2, b2.reshape(1, C))


def _scale_body(x_ref, s_ref, out_ref):
    out_ref[...] = x_ref[...] * s_ref[...]


def _scale(x2, scales2):
    return pl.pallas_call(
        _scale_body,
        grid=(N_ROW, N_COL),
        in_specs=[
            pl.BlockSpec((ROW_BLK, COL_BLK), lambda i, j: (i, j)),
            pl.BlockSpec((ROW_BLK, 1), lambda i, j: (i, 0)),
        ],
        out_specs=pl.BlockSpec((ROW_BLK, COL_BLK), lambda i, j: (i, j)),
        out_shape=jax.ShapeDtypeStruct((ROWS, HW), jnp.float32),
    )(x2, scales2)


def kernel(x, prototypes, W1, b1, W2, b2):
    x2 = x.reshape(ROWS, HW)
    summary = _summary(x2).reshape(B, C)
    d2 = _distances(summary, prototypes)
    matched = _argmin_gather(d2, prototypes)
    scales = _mlp(matched, W1, b1, W2, b2)
    out2 = _scale(x2, scales.reshape(ROWS, 1))
    return out2.reshape(B, C, H, W)
